# Initial kernel scaffold; baseline (speedup 1.0000x reference)
import jax
import jax.numpy as jnp
from jax.experimental import pallas as pl


def kernel(z, codebook):
    raise NotImplementedError("write your pallas kernel here")


# speedup vs baseline: 1.2304x; 1.2304x over previous
import jax
import jax.numpy as jnp
from jax.experimental import pallas as pl
from jax.experimental.pallas import tpu as pltpu

_B = 8
_D = 256
_K = 8192
_T = 576
_KC = 1024
_NKC = _K // _KC
_N = _B * _T


def _vq_body(z_ref, cb_ref, oh_ref, zq_ref, idx_ref, ihist_ref, shist_ref,
             stats_ref, d_scr, zq_scr, hist_scr, ms_scr, idx_scr):
    b = pl.program_id(0)
    kc = pl.program_id(1)

    @pl.when(kc == 0)
    def _pass1():
        z = z_ref[0]
        z2 = jnp.sum(z * z, axis=1, keepdims=True)
        minv = None
        idx = None
        for c in range(_NKC):
            sl = slice(c * _KC, (c + 1) * _KC)
            cb = cb_ref[sl, :]
            c2 = jnp.sum(cb * cb, axis=1)
            zc = jax.lax.dot_general(z, cb, (((1,), (1,)), ((), ())),
                                     preferred_element_type=jnp.float32)
            d = (z2 + c2[None, :]) - 2.0 * zc
            d_scr[c] = d
            mc = jnp.min(d, axis=1, keepdims=True)
            lane = jax.lax.broadcasted_iota(jnp.int32, (_T, _KC), 1)
            ic = jnp.min(jnp.where(d == mc, lane + c * _KC, _K), axis=1,
                         keepdims=True)
            if c == 0:
                minv, idx = mc, ic
            else:
                upd = mc < minv
                idx = jnp.where(upd, ic, idx)
                minv = jnp.where(upd, mc, minv)

        s = jnp.zeros((_T, 1), jnp.float32)
        for c in range(_NKC):
            s = s + jnp.sum(jnp.exp(minv - d_scr[c]), axis=1, keepdims=True)

        ms_scr[:, 0:1] = minv
        ms_scr[:, 1:2] = 1.0 / s
        idx_flat = idx[:, 0]
        idx_scr[0, :] = idx_flat
        idx_ref[0, 0, :] = idx_flat

        sum_dmin = jnp.sum(minv)
        sum_logs = jnp.sum(jnp.log(s))
        lane128 = jax.lax.broadcasted_iota(jnp.int32, (1, 128), 1)
        stats_ref[0] = (jnp.where(lane128 == 0, sum_dmin, 0.0)
                        + jnp.where(lane128 == 1, sum_logs, 0.0))

    minv = ms_scr[:, 0:1]
    inv_s = ms_scr[:, 1:2]
    p = jnp.exp(minv - d_scr[kc]) * inv_s
    shist_ref[0, kc, 0, :] = jnp.sum(p, axis=0)

    idx_row = idx_scr[0, :]
    kio = jax.lax.broadcasted_iota(jnp.int32, (_KC, _T), 0) + kc * _KC
    oh = (kio == idx_row[None, :]).astype(jnp.float32)
    oh_ref[0] = oh
    ih = jnp.sum(oh, axis=1)
    ihist_ref[0, kc, 0, :] = ih

    hist_scr[kc] = jnp.where(b == 0, ih, hist_scr[kc] + ih)

    cbc = cb_ref[pl.ds(kc * _KC, _KC), :]
    zq_c = jax.lax.dot_general(cbc, oh, (((0,), (0,)), ((), ())),
                               preferred_element_type=jnp.float32)
    acc = jnp.where(kc == 0, zq_c, zq_scr[...] + zq_c)
    zq_scr[...] = acc

    @pl.when(kc == _NKC - 1)
    def _finish():
        zq_ref[0] = zq_scr[...]
        pr = hist_scr[...] * (1.0 / _N)
        ent = jnp.sum(pr * jnp.log(pr + 1e-10))
        lane128 = jax.lax.broadcasted_iota(jnp.int32, (1, 128), 1)
        stats_ref[0] = stats_ref[0] + jnp.where(lane128 == 2, ent, 0.0)


def kernel(z, codebook):
    z_bt = jnp.transpose(z, (0, 2, 3, 1)).reshape(_B, _T, _D)
    oh, zq, idxo, ihist, shist, stats = pl.pallas_call(
        _vq_body,
        grid=(_B, _NKC),
        in_specs=[
            pl.BlockSpec((1, _T, _D), lambda b, kc: (b, 0, 0)),
            pl.BlockSpec((_K, _D), lambda b, kc: (0, 0)),
        ],
        out_specs=[
            pl.BlockSpec((1, _KC, _T), lambda b, kc: (b, kc, 0)),
            pl.BlockSpec((1, _D, _T), lambda b, kc: (b, 0, 0)),
            pl.BlockSpec((1, 1, _T), lambda b, kc: (b, 0, 0)),
            pl.BlockSpec((1, _NKC, 1, _KC), lambda b, kc: (b, 0, 0, 0)),
            pl.BlockSpec((1, _NKC, 1, _KC), lambda b, kc: (b, 0, 0, 0)),
            pl.BlockSpec((1, 1, 128), lambda b, kc: (b, 0, 0)),
        ],
        out_shape=[
            jax.ShapeDtypeStruct((_B, _K, _T), jnp.float32),
            jax.ShapeDtypeStruct((_B, _D, _T), jnp.float32),
            jax.ShapeDtypeStruct((_B, 1, _T), jnp.int32),
            jax.ShapeDtypeStruct((_B, _NKC, 1, _KC), jnp.float32),
            jax.ShapeDtypeStruct((_B, _NKC, 1, _KC), jnp.float32),
            jax.ShapeDtypeStruct((_B, 1, 128), jnp.float32),
        ],
        scratch_shapes=[
            pltpu.VMEM((_NKC, _T, _KC), jnp.float32),
            pltpu.VMEM((_D, _T), jnp.float32),
            pltpu.VMEM((_NKC, _KC), jnp.float32),
            pltpu.VMEM((_T, 128), jnp.float32),
            pltpu.VMEM((8, _T), jnp.int32),
        ],
    )(z_bt, codebook)

    dmin_tot = jnp.sum(stats[:, 0, 0])
    logs_tot = jnp.sum(stats[:, 0, 1])
    ent = stats[_B - 1, 0, 2]
    closs = dmin_tot / (_N * _D)
    loss = 1.25 * closs
    sloss = logs_tot / _N
    perp = jnp.exp(-ent)
    z_q_ste = zq.reshape(_B, _D, 24, 24)
    onehot_out = oh.reshape(_B, _K, 24, 24)
    idx_out = idxo.reshape(_B, 1, 24, 24)
    index_histogram = ihist.reshape(_B, _K)
    softmax_histogram = shist.reshape(_B, _K)
    return (loss, z_q_ste, perp, onehot_out, idx_out, index_histogram,
            softmax_histogram, closs, closs, sloss)
